# baseline (device time: 69307 ns/iter reference)
import jax
import jax.numpy as jnp
from jax import lax
from jax.experimental import pallas as pl
from jax.experimental.pallas import tpu as pltpu


def kernel(Q, K, V):
    b, s, h, d = Q.shape
    hd = h * d
    qr_rows = s // 4
    scale = d ** -0.5
    f32 = jnp.float32
    bf16 = jnp.bfloat16
    HPC = 2
    nch = h // HPC

    NT = (((1,), (1,)), ((), ()))
    NN = (((1,), (0,)), ((), ()))

    def body(q_ref, k_ref, v_ref, o_ref, qr, obz, oq0, ob, lz, lq0,
             qz_s, qz_r, oz_s, oz_r, xg_s, xg_r, yg_s, yg_r):
        my_x = lax.axis_index("x")
        my_y = lax.axis_index("y")
        my_z = lax.axis_index("z")
        zpeer = (my_x, my_y, 1 - my_z)
        xpeer = (1 - my_x, my_y, my_z)
        ypeer = (my_x, 1 - my_y, my_z)

        barrier = pltpu.get_barrier_semaphore()
        for peer in (zpeer, xpeer, ypeer):
            pl.semaphore_signal(
                barrier, inc=1, device_id=peer,
                device_id_type=pl.DeviceIdType.MESH,
            )
        pl.semaphore_wait(barrier, 3)

        p_me = 2 * my_x + my_y
        p_x = 2 * (1 - my_x) + my_y
        p_y = 2 * my_x + (1 - my_y)
        p_d = 2 * (1 - my_x) + (1 - my_y)

        def cols(head):
            return pl.ds(head * d, d)

        def ccols(chunk):
            return pl.ds(chunk * HPC * d, HPC * d)

        def rows(pidx):
            return pl.ds(pidx * qr_rows, qr_rows)

        def rdma(src, dst, ssem, rsem, peer):
            r = pltpu.make_async_remote_copy(
                src_ref=src, dst_ref=dst, send_sem=ssem, recv_sem=rsem,
                device_id=peer, device_id_type=pl.DeviceIdType.MESH,
            )
            r.start()
            return r

        started = []

        qz = []
        for i in range(nch):
            qz.append(rdma(q_ref.at[rows(p_me), ccols(i)],
                           qr.at[:, ccols(i)],
                           qz_s.at[i], qz_r.at[i], zpeer))
        started += qz

        oz = []
        for i in range(nch):
            qz[i].wait_recv()
            for j in range(HPC * i, HPC * (i + 1)):
                c = cols(j)
                q2 = qr[:, c]
                s2 = lax.dot_general(q2, k_ref[:, c], NT,
                                     preferred_element_type=f32)
                p2 = jnp.exp(s2.astype(bf16))
                lz[:, j:j + 1] = jnp.sum(p2, axis=-1, keepdims=True,
                                         dtype=f32)
                o2 = lax.dot_general(p2, v_ref[:, c], NN,
                                     preferred_element_type=f32)
                obz[:, c] = o2.astype(bf16)
            oz.append(rdma(obz.at[:, ccols(i)], oq0.at[:, ccols(i)],
                           oz_s.at[i], oz_r.at[i], zpeer))
        oz_l = rdma(lz, lq0, oz_s.at[nch], oz_r.at[nch], zpeer)
        started += oz
        started.append(oz_l)
        oz_l.wait_recv()

        xg, yg = [], []
        mrows = rows(p_me)
        for i in range(nch):
            oz[i].wait_recv()
            for j in range(HPC * i, HPC * (i + 1)):
                c = cols(j)
                q1 = q_ref[mrows, c]
                s1 = lax.dot_general(q1, k_ref[:, c], NT,
                                     preferred_element_type=f32)
                p1 = jnp.exp(s1.astype(bf16))
                l1 = jnp.sum(p1, axis=-1, keepdims=True, dtype=f32)
                o1 = lax.dot_general(p1, v_ref[:, c], NN,
                                     preferred_element_type=f32)
                o = (o1 + oq0[:, c].astype(f32)) / (l1 + lq0[:, j:j + 1])
                o_ref[mrows, c] = o
                ob[mrows, c] = o.astype(bf16)
            xg.append(rdma(ob.at[mrows, ccols(i)], ob.at[mrows, ccols(i)],
                           xg_s.at[i], xg_r.at[i], xpeer))
            yg.append(rdma(ob.at[mrows, ccols(i)], ob.at[mrows, ccols(i)],
                           yg_s.at[i], yg_r.at[i], ypeer))
        started += xg + yg

        xd, yd = [], []
        for i in range(nch):
            c2 = ccols(i)
            xg[i].wait_recv()
            if i % 2 == 1:
                yd.append(rdma(ob.at[rows(p_x), c2], ob.at[rows(p_x), c2],
                               yg_s.at[nch + i // 2], yg_r.at[nch + i // 2],
                               ypeer))
            o_ref[rows(p_x), c2] = ob[rows(p_x), c2].astype(f32)
            yg[i].wait_recv()
            if i % 2 == 0:
                xd.append(rdma(ob.at[rows(p_y), c2], ob.at[rows(p_y), c2],
                               xg_s.at[nch + i // 2], xg_r.at[nch + i // 2],
                               xpeer))
            o_ref[rows(p_y), c2] = ob[rows(p_y), c2].astype(f32)
        started += xd + yd

        for i in range(nch):
            c2 = ccols(i)
            if i % 2 == 0:
                xd[i // 2].wait_recv()
            else:
                yd[i // 2].wait_recv()
            o_ref[rows(p_d), c2] = ob[rows(p_d), c2].astype(f32)

        for r in started:
            r.wait_send()

    out = pl.pallas_call(
        body,
        out_shape=jax.ShapeDtypeStruct((s, hd), f32),
        in_specs=[pl.BlockSpec(memory_space=pltpu.MemorySpace.VMEM)] * 3,
        out_specs=pl.BlockSpec(memory_space=pltpu.MemorySpace.VMEM),
        scratch_shapes=[
            pltpu.VMEM((qr_rows, hd), bf16),
            pltpu.VMEM((qr_rows, hd), bf16),
            pltpu.VMEM((qr_rows, hd), bf16),
            pltpu.VMEM((s, hd), bf16),
            pltpu.VMEM((qr_rows, h), f32),
            pltpu.VMEM((qr_rows, h), f32),
            pltpu.SemaphoreType.DMA((nch,)),
            pltpu.SemaphoreType.DMA((nch,)),
            pltpu.SemaphoreType.DMA((nch + 1,)),
            pltpu.SemaphoreType.DMA((nch + 1,)),
            pltpu.SemaphoreType.DMA((nch + nch // 2,)),
            pltpu.SemaphoreType.DMA((nch + nch // 2,)),
            pltpu.SemaphoreType.DMA((nch + nch // 2,)),
            pltpu.SemaphoreType.DMA((nch + nch // 2,)),
        ],
        compiler_params=pltpu.CompilerParams(
            collective_id=0, has_side_effects=True
        ),
    )(
        (Q.reshape(s, hd) * scale).astype(bf16),
        K.reshape(s, hd).astype(bf16),
        V.reshape(s, hd).astype(bf16),
    )
    return out.reshape(b, s, h, d)
